# Initial kernel scaffold; baseline (speedup 1.0000x reference)
import jax
import jax.numpy as jnp
from jax.experimental import pallas as pl


def kernel(label_input, word_embeddings, ln_weight, ln_bias):
    raise NotImplementedError("write your pallas kernel here")


# speedup vs baseline: 6.3505x; 6.3505x over previous
import jax
import jax.numpy as jnp
from jax.experimental import pallas as pl


_EPS = 1e-12
_ROWS_PER_BLOCK = 32


def _body(lab_ref, emb_ref, w_ref, b_ref, out_ref):
    tab = emb_ref[...]
    mean = jnp.mean(tab, axis=-1, keepdims=True)
    var = jnp.mean(jnp.square(tab - mean), axis=-1, keepdims=True)
    nt = (tab - mean) / jnp.sqrt(var + _EPS) * w_ref[...][None, :] + b_ref[...][None, :]
    nt0 = nt[0, :]
    diff = nt[1, :] - nt[0, :]
    lab = lab_ref[...].astype(jnp.float32)
    out_ref[...] = nt0[None, None, :] + lab[:, :, None] * diff[None, None, :]


def kernel(label_input, word_embeddings, ln_weight, ln_bias):
    n, s = label_input.shape
    v, d = word_embeddings.shape
    grid = (n // _ROWS_PER_BLOCK,)
    return pl.pallas_call(
        _body,
        grid=grid,
        in_specs=[
            pl.BlockSpec((_ROWS_PER_BLOCK, s), lambda i: (i, 0)),
            pl.BlockSpec((v, d), lambda i: (0, 0)),
            pl.BlockSpec((d,), lambda i: (0,)),
            pl.BlockSpec((d,), lambda i: (0,)),
        ],
        out_specs=pl.BlockSpec((_ROWS_PER_BLOCK, s, d), lambda i: (i, 0, 0)),
        out_shape=jax.ShapeDtypeStruct((n, s, d), jnp.float32),
    )(label_input, word_embeddings, ln_weight, ln_bias)
